# baseline (speedup 1.0000x reference)
import functools

import jax
import jax.numpy as jnp
from jax import lax
from jax.experimental import pallas as pl
from jax.experimental.pallas import tpu as pltpu
from jax.experimental.pallas import tpu_sc as plsc

EMBED = 128
ROWS = 4096
WIDTH = 13 * EMBED
CHUNK = 4
NBUF = 4
DEPTH = 2

_PLANS = []
for _start in (0, 1):
    _plan = []
    for _j, _i in enumerate(range(_start, 26, 2)):
        _plan.append((0 if _i < 13 else 1, (_i % 13) * EMBED, _j * EMBED))
    _PLANS.append(tuple(_plan))
_PLANS = tuple(_PLANS)


def _make_sc_odd():
    info = plsc.get_sparse_core_info()
    nc, ns = info.num_cores, info.num_subcores
    nw = nc * ns
    rpw = ROWS // nw
    nchunks = rpw // CHUNK

    mesh = plsc.VectorSubcoreMesh(core_axis_name="c", subcore_axis_name="s")

    @functools.partial(
        pl.kernel,
        mesh=mesh,
        out_type=[jax.ShapeDtypeStruct((ROWS, WIDTH), jnp.float32)],
        scratch_types=(
            [pltpu.VMEM((CHUNK, WIDTH), jnp.float32) for _ in range(NBUF)]
            + [pltpu.SemaphoreType.DMA for _ in range(DEPTH + NBUF)]
        ),
    )
    def regroup_odd(kt0, kt1, out_odd, *scratch):
        bufs = scratch[:NBUF]
        gsems = scratch[NBUF : NBUF + DEPTH]
        ssems = scratch[NBUF + DEPTH :]
        wid = lax.axis_index("s") * nc + lax.axis_index("c")
        r0 = wid * rpw
        srcs = (kt0, kt1)
        nit = nchunks

        def fire_gathers(c):
            rows = r0 + c * CHUNK
            buf = bufs[c % NBUF]
            sem = gsems[c % DEPTH]
            handles = []
            for src, sc, dc in _PLANS[1]:
                handles.append(
                    pltpu.async_copy(
                        srcs[src].at[pl.ds(rows, CHUNK), pl.ds(sc, EMBED)],
                        buf.at[:, pl.ds(dc, EMBED)],
                        sem,
                    )
                )
            return handles

        gh = {}
        for k in range(min(DEPTH, nit)):
            gh[k] = fire_gathers(k)
        sh = [None] * nit
        for it in range(nit):
            for h in gh.pop(it):
                h.wait()
            rows = r0 + it * CHUNK
            sh[it] = pltpu.async_copy(
                bufs[it % NBUF], out_odd.at[pl.ds(rows, CHUNK), :], ssems[it % NBUF]
            )
            nx = it + DEPTH
            if nx < nit:
                if nx - NBUF >= 0:
                    sh[nx - NBUF].wait()
                gh[nx] = fire_gathers(nx)
        for j in range(max(0, nit - NBUF), nit):
            sh[j].wait()

    return regroup_odd


def _tc_even_body(kt0_ref, kt1_ref, out_ref):
    j = pl.program_id(1)

    @pl.when(j < 7)
    def _():
        out_ref[...] = kt0_ref[...]

    @pl.when(j >= 7)
    def _():
        out_ref[...] = kt1_ref[...]


def _tc_even(kt0, kt1):
    return pl.pallas_call(
        _tc_even_body,
        grid=(1, 13),
        in_specs=[
            pl.BlockSpec((ROWS, EMBED), lambda r, j: (r, jnp.minimum(2 * j, 12))),
            pl.BlockSpec(
                (ROWS, EMBED), lambda r, j: (r, jnp.maximum(2 * j - 13, 1))
            ),
        ],
        out_specs=pl.BlockSpec((ROWS, EMBED), lambda r, j: (r, j)),
        out_shape=jax.ShapeDtypeStruct((ROWS, WIDTH), jnp.float32),
    )(kt0, kt1)


_SC_ODD = None


def kernel(kt0_values, kt1_values):
    global _SC_ODD
    if _SC_ODD is None:
        _SC_ODD = _make_sc_odd()
    (odd,) = _SC_ODD(kt0_values, kt1_values)
    even = _tc_even(kt0_values, kt1_values)
    return (even, odd)


# speedup vs baseline: 1.0729x; 1.0729x over previous
import functools

import jax
import jax.numpy as jnp
from jax import lax
from jax.experimental import pallas as pl
from jax.experimental.pallas import tpu as pltpu
from jax.experimental.pallas import tpu_sc as plsc

EMBED = 128
ROWS = 4096
WIDTH = 13 * EMBED
CHUNK = 8
NBUF = 6
DEPTH = 3

_PLANS = []
for _start in (0, 1):
    _plan = []
    for _j, _i in enumerate(range(_start, 26, 2)):
        _plan.append((0 if _i < 13 else 1, (_i % 13) * EMBED, _j * EMBED))
    _PLANS.append(tuple(_plan))
_PLANS = tuple(_PLANS)


def _make_sc_odd():
    info = plsc.get_sparse_core_info()
    nc, ns = info.num_cores, info.num_subcores
    nw = nc * ns
    rpw = ROWS // nw
    nchunks = rpw // CHUNK

    mesh = plsc.VectorSubcoreMesh(core_axis_name="c", subcore_axis_name="s")

    @functools.partial(
        pl.kernel,
        mesh=mesh,
        out_type=[jax.ShapeDtypeStruct((ROWS, WIDTH), jnp.float32)],
        scratch_types=(
            [pltpu.VMEM((CHUNK, WIDTH), jnp.float32) for _ in range(NBUF)]
            + [pltpu.SemaphoreType.DMA for _ in range(DEPTH + NBUF)]
        ),
    )
    def regroup_odd(kt0, kt1, out_odd, *scratch):
        bufs = scratch[:NBUF]
        gsems = scratch[NBUF : NBUF + DEPTH]
        ssems = scratch[NBUF + DEPTH :]
        wid = lax.axis_index("s") * nc + lax.axis_index("c")
        r0 = wid * rpw
        srcs = (kt0, kt1)
        nit = nchunks

        def fire_gathers(c):
            rows = r0 + c * CHUNK
            buf = bufs[c % NBUF]
            sem = gsems[c % DEPTH]
            handles = []
            for src, sc, dc in _PLANS[1]:
                handles.append(
                    pltpu.async_copy(
                        srcs[src].at[pl.ds(rows, CHUNK), pl.ds(sc, EMBED)],
                        buf.at[:, pl.ds(dc, EMBED)],
                        sem,
                    )
                )
            return handles

        gh = {}
        for k in range(min(DEPTH, nit)):
            gh[k] = fire_gathers(k)
        sh = [None] * nit
        for it in range(nit):
            for h in gh.pop(it):
                h.wait()
            rows = r0 + it * CHUNK
            sh[it] = pltpu.async_copy(
                bufs[it % NBUF], out_odd.at[pl.ds(rows, CHUNK), :], ssems[it % NBUF]
            )
            nx = it + DEPTH
            if nx < nit:
                if nx - NBUF >= 0:
                    sh[nx - NBUF].wait()
                gh[nx] = fire_gathers(nx)
        for j in range(max(0, nit - NBUF), nit):
            sh[j].wait()

    return regroup_odd


def _tc_even_body(kt0_ref, kt1_ref, out_ref):
    j = pl.program_id(1)

    @pl.when(j < 7)
    def _():
        out_ref[...] = kt0_ref[...]

    @pl.when(j >= 7)
    def _():
        out_ref[...] = kt1_ref[...]


def _tc_even(kt0, kt1):
    return pl.pallas_call(
        _tc_even_body,
        grid=(1, 13),
        in_specs=[
            pl.BlockSpec((ROWS, EMBED), lambda r, j: (r, jnp.minimum(2 * j, 12))),
            pl.BlockSpec(
                (ROWS, EMBED), lambda r, j: (r, jnp.maximum(2 * j - 13, 1))
            ),
        ],
        out_specs=pl.BlockSpec((ROWS, EMBED), lambda r, j: (r, j)),
        out_shape=jax.ShapeDtypeStruct((ROWS, WIDTH), jnp.float32),
    )(kt0, kt1)


_SC_ODD = None


def kernel(kt0_values, kt1_values):
    global _SC_ODD
    if _SC_ODD is None:
        _SC_ODD = _make_sc_odd()
    (odd,) = _SC_ODD(kt0_values, kt1_values)
    even = _tc_even(kt0_values, kt1_values)
    return (even, odd)


# speedup vs baseline: 1.0764x; 1.0032x over previous
import functools

import jax
import jax.numpy as jnp
from jax import lax
from jax.experimental import pallas as pl
from jax.experimental.pallas import tpu as pltpu
from jax.experimental.pallas import tpu_sc as plsc

EMBED = 128
ROWS = 4096
WIDTH = 13 * EMBED
CHUNK = 8
NBUF = 8
DEPTH = 4

_PLANS = []
for _start in (0, 1):
    _plan = []
    for _j, _i in enumerate(range(_start, 26, 2)):
        _plan.append((0 if _i < 13 else 1, (_i % 13) * EMBED, _j * EMBED))
    _PLANS.append(tuple(_plan))
_PLANS = tuple(_PLANS)


def _make_sc_odd():
    info = plsc.get_sparse_core_info()
    nc, ns = info.num_cores, info.num_subcores
    nw = nc * ns
    rpw = ROWS // nw
    nchunks = rpw // CHUNK

    mesh = plsc.VectorSubcoreMesh(core_axis_name="c", subcore_axis_name="s")

    @functools.partial(
        pl.kernel,
        mesh=mesh,
        out_type=[jax.ShapeDtypeStruct((ROWS, WIDTH), jnp.float32)],
        scratch_types=(
            [pltpu.VMEM((CHUNK, WIDTH), jnp.float32) for _ in range(NBUF)]
            + [pltpu.SemaphoreType.DMA for _ in range(DEPTH + NBUF)]
        ),
    )
    def regroup_odd(kt0, kt1, out_odd, *scratch):
        bufs = scratch[:NBUF]
        gsems = scratch[NBUF : NBUF + DEPTH]
        ssems = scratch[NBUF + DEPTH :]
        wid = lax.axis_index("s") * nc + lax.axis_index("c")
        r0 = wid * rpw
        srcs = (kt0, kt1)
        nit = nchunks

        def fire_gathers(c):
            rows = r0 + c * CHUNK
            buf = bufs[c % NBUF]
            sem = gsems[c % DEPTH]
            handles = []
            for src, sc, dc in _PLANS[1]:
                handles.append(
                    pltpu.async_copy(
                        srcs[src].at[pl.ds(rows, CHUNK), pl.ds(sc, EMBED)],
                        buf.at[:, pl.ds(dc, EMBED)],
                        sem,
                    )
                )
            return handles

        gh = {}
        for k in range(min(DEPTH, nit)):
            gh[k] = fire_gathers(k)
        sh = [None] * nit
        for it in range(nit):
            for h in gh.pop(it):
                h.wait()
            rows = r0 + it * CHUNK
            sh[it] = pltpu.async_copy(
                bufs[it % NBUF], out_odd.at[pl.ds(rows, CHUNK), :], ssems[it % NBUF]
            )
            nx = it + DEPTH
            if nx < nit:
                if nx - NBUF >= 0:
                    sh[nx - NBUF].wait()
                gh[nx] = fire_gathers(nx)
        for j in range(max(0, nit - NBUF), nit):
            sh[j].wait()

    return regroup_odd


def _tc_even_body(kt0_ref, kt1_ref, out_ref):
    j = pl.program_id(1)

    @pl.when(j < 7)
    def _():
        out_ref[...] = kt0_ref[...]

    @pl.when(j >= 7)
    def _():
        out_ref[...] = kt1_ref[...]


def _tc_even(kt0, kt1):
    return pl.pallas_call(
        _tc_even_body,
        grid=(1, 13),
        in_specs=[
            pl.BlockSpec((ROWS, EMBED), lambda r, j: (r, jnp.minimum(2 * j, 12))),
            pl.BlockSpec(
                (ROWS, EMBED), lambda r, j: (r, jnp.maximum(2 * j - 13, 1))
            ),
        ],
        out_specs=pl.BlockSpec((ROWS, EMBED), lambda r, j: (r, j)),
        out_shape=jax.ShapeDtypeStruct((ROWS, WIDTH), jnp.float32),
    )(kt0, kt1)


_SC_ODD = None


def kernel(kt0_values, kt1_values):
    global _SC_ODD
    if _SC_ODD is None:
        _SC_ODD = _make_sc_odd()
    (odd,) = _SC_ODD(kt0_values, kt1_values)
    even = _tc_even(kt0_values, kt1_values)
    return (even, odd)
